# initial kernel scaffold (unmeasured)
import jax
import jax.numpy as jnp
from jax import lax
from jax.experimental import pallas as pl
from jax.experimental.pallas import tpu as pltpu

N_DEV = 4


def kernel(x, router_W, route_idx, expert_W):
    n, d = x.shape
    e_loc, _, h = expert_W.shape
    n_exp = router_W.shape[1]
    chunk = n // N_DEV

    def body(x_ref, rw_ref, idx_ref, ew_ref, out_ref,
             part_ref, send_ref, recv_ref, send_sems, recv_sems):
        my = lax.axis_index("i")
        left = lax.rem(my + N_DEV - 1, N_DEV)
        right = lax.rem(my + 1, N_DEV)

        barrier = pltpu.get_barrier_semaphore()
        pl.semaphore_signal(barrier, inc=1, device_id=(left,),
                            device_id_type=pl.DeviceIdType.MESH)
        pl.semaphore_signal(barrier, inc=1, device_id=(right,),
                            device_id_type=pl.DeviceIdType.MESH)
        pl.semaphore_wait(barrier, 2)

        xv = x_ref[:, :]
        scores = jnp.dot(xv, rw_ref[:, :], preferred_element_type=jnp.float32)
        probs = jax.nn.softmax(scores, axis=-1)
        e0 = idx_ref[:, 0:1]
        e1 = idx_ref[:, 1:2]
        iota = lax.broadcasted_iota(jnp.int32, (n, n_exp), 1)
        p0 = jnp.sum(jnp.where(iota == e0, probs, 0.0), axis=1, keepdims=True)
        p1 = jnp.sum(jnp.where(iota == e1, probs, 0.0), axis=1, keepdims=True)
        denom = p0 + p1

        acc = jnp.zeros((n, h), jnp.float32)
        for j in range(e_loc):
            g = my * e_loc + j
            gate = (jnp.where(e0 == g, p0, 0.0)
                    + jnp.where(e1 == g, p1, 0.0)) / denom
            acc = acc + jnp.dot(xv * gate, ew_ref[j],
                                preferred_element_type=jnp.float32)
        part_ref[:, :] = acc

        for s in range(N_DEV - 1):
            c_send = lax.rem(my + N_DEV - s - 1, N_DEV)
            if s == 0:
                send_ref[:, :] = part_ref[pl.ds(c_send * chunk, chunk), :]
            rdma = pltpu.make_async_remote_copy(
                src_ref=send_ref,
                dst_ref=recv_ref.at[s],
                send_sem=send_sems.at[s],
                recv_sem=recv_sems.at[s],
                device_id=(right,),
                device_id_type=pl.DeviceIdType.MESH,
            )
            rdma.start()
            rdma.wait()
            c_recv = lax.rem(my + 2 * N_DEV - s - 2, N_DEV)
            summed = recv_ref[s] + part_ref[pl.ds(c_recv * chunk, chunk), :]
            if s < N_DEV - 2:
                send_ref[:, :] = summed
            else:
                out_ref[:, :] = summed

    return pl.pallas_call(
        body,
        out_shape=jax.ShapeDtypeStruct((chunk, h), jnp.float32),
        in_specs=[pl.BlockSpec(memory_space=pltpu.VMEM)] * 4,
        out_specs=pl.BlockSpec(memory_space=pltpu.VMEM),
        scratch_shapes=[
            pltpu.VMEM((n, h), jnp.float32),
            pltpu.VMEM((chunk, h), jnp.float32),
            pltpu.VMEM((N_DEV - 1, chunk, h), jnp.float32),
            pltpu.SemaphoreType.DMA((N_DEV - 1,)),
            pltpu.SemaphoreType.DMA((N_DEV - 1,)),
        ],
        compiler_params=pltpu.CompilerParams(collective_id=0),
    )(x, router_W, route_idx, expert_W)


# baseline (device time: 120088 ns/iter reference)
import jax
import jax.numpy as jnp
from jax import lax
from jax.experimental import pallas as pl
from jax.experimental.pallas import tpu as pltpu

N_DEV = 4


def kernel(x, router_W, route_idx, expert_W):
    n, d = x.shape
    e_loc, _, h = expert_W.shape
    n_exp = router_W.shape[1]
    chunk = n // N_DEV

    def body(x_ref, rw_ref, idx_ref, ew_ref, out_ref,
             part_ref, send_ref, recv_ref, send_sems, recv_sems):
        my = lax.axis_index("i")
        left = lax.rem(my + N_DEV - 1, N_DEV)
        right = lax.rem(my + 1, N_DEV)

        barrier = pltpu.get_barrier_semaphore()
        pl.semaphore_signal(barrier, inc=1, device_id=(left,),
                            device_id_type=pl.DeviceIdType.MESH)
        pl.semaphore_signal(barrier, inc=1, device_id=(right,),
                            device_id_type=pl.DeviceIdType.MESH)
        pl.semaphore_wait(barrier, 2)

        xv = x_ref[:, :]
        scores = jnp.dot(xv, rw_ref[:, :], preferred_element_type=jnp.float32)
        probs = jax.nn.softmax(scores, axis=-1)
        e0 = idx_ref[:, 0:1]
        e1 = idx_ref[:, 1:2]
        iota = lax.broadcasted_iota(jnp.int32, (n, n_exp), 1)
        p0 = jnp.sum(jnp.where(iota == e0, probs, 0.0), axis=1, keepdims=True)
        p1 = jnp.sum(jnp.where(iota == e1, probs, 0.0), axis=1, keepdims=True)
        denom = p0 + p1

        for j in range(e_loc):
            g = my * e_loc + j
            gate = (jnp.where(e0 == g, p0, 0.0)
                    + jnp.where(e1 == g, p1, 0.0)) / denom
            contrib = jnp.dot(xv * gate, ew_ref[j],
                              preferred_element_type=jnp.float32)
            if j == 0:
                part_ref[:, :] = contrib
            else:
                part_ref[:, :] = part_ref[:, :] + contrib

        for s in range(N_DEV - 1):
            c_send = lax.rem(my + N_DEV - s - 1, N_DEV)
            if s == 0:
                send_ref[:, :] = part_ref[pl.ds(c_send * chunk, chunk), :]
            rdma = pltpu.make_async_remote_copy(
                src_ref=send_ref,
                dst_ref=recv_ref.at[s],
                send_sem=send_sems.at[s],
                recv_sem=recv_sems.at[s],
                device_id=(right,),
                device_id_type=pl.DeviceIdType.MESH,
            )
            rdma.start()
            rdma.wait()
            c_recv = lax.rem(my + 2 * N_DEV - s - 2, N_DEV)
            summed = recv_ref[s] + part_ref[pl.ds(c_recv * chunk, chunk), :]
            if s < N_DEV - 2:
                send_ref[:, :] = summed
            else:
                out_ref[:, :] = summed

    return pl.pallas_call(
        body,
        out_shape=jax.ShapeDtypeStruct((chunk, h), jnp.float32),
        in_specs=[pl.BlockSpec(memory_space=pltpu.VMEM)] * 4,
        out_specs=pl.BlockSpec(memory_space=pltpu.VMEM),
        scratch_shapes=[
            pltpu.VMEM((n, h), jnp.float32),
            pltpu.VMEM((chunk, h), jnp.float32),
            pltpu.VMEM((N_DEV - 1, chunk, h), jnp.float32),
            pltpu.SemaphoreType.DMA((N_DEV - 1,)),
            pltpu.SemaphoreType.DMA((N_DEV - 1,)),
        ],
        compiler_params=pltpu.CompilerParams(
            collective_id=0,
            vmem_limit_bytes=100 * 1024 * 1024,
        ),
    )(x, router_W, route_idx, expert_W)


# device time: 98001 ns/iter; 1.2254x vs baseline; 1.2254x over previous
import jax
import jax.numpy as jnp
from jax import lax
from jax.experimental import pallas as pl
from jax.experimental.pallas import tpu as pltpu

N_DEV = 4


def kernel(x, router_W, route_idx, expert_W):
    n, d = x.shape
    e_loc, _, h = expert_W.shape
    n_exp = router_W.shape[1]
    chunk = n // N_DEV

    def body(x_ref, rw_ref, idx_ref, ew_ref, out_ref,
             part_ref, recv_ref, send_sems, recv_sems):
        my = lax.axis_index("i")
        left = lax.rem(my + N_DEV - 1, N_DEV)
        right = lax.rem(my + 1, N_DEV)

        barrier = pltpu.get_barrier_semaphore()
        pl.semaphore_signal(barrier, inc=1, device_id=(left,),
                            device_id_type=pl.DeviceIdType.MESH)
        pl.semaphore_signal(barrier, inc=1, device_id=(right,),
                            device_id_type=pl.DeviceIdType.MESH)
        pl.semaphore_wait(barrier, 2)

        lids = my * e_loc + lax.broadcasted_iota(jnp.int32, (1, e_loc), 1)
        iota = lax.broadcasted_iota(jnp.int32, (chunk, n_exp), 1)

        def compute_chunk(c):
            off = c * chunk
            xc = x_ref[pl.ds(off, chunk), :]
            scores = jnp.dot(xc, rw_ref[:, :],
                             preferred_element_type=jnp.float32)
            probs = jax.nn.softmax(scores, axis=-1)
            e0c = idx_ref[pl.ds(off, chunk), 0:1]
            e1c = idx_ref[pl.ds(off, chunk), 1:2]
            p0c = jnp.sum(jnp.where(iota == e0c, probs, 0.0),
                          axis=1, keepdims=True)
            p1c = jnp.sum(jnp.where(iota == e1c, probs, 0.0),
                          axis=1, keepdims=True)
            gates = (jnp.where(e0c == lids, p0c, 0.0)
                     + jnp.where(e1c == lids, p1c, 0.0)) / (p0c + p1c)
            acc = jnp.dot(xc * gates[:, 0:1], ew_ref[0],
                          preferred_element_type=jnp.float32)
            for j in range(1, e_loc):
                acc = acc + jnp.dot(xc * gates[:, j:j + 1], ew_ref[j],
                                    preferred_element_type=jnp.float32)
            part_ref[pl.ds(off, chunk), :] = acc

        c0 = lax.rem(my + 3, N_DEV)
        c1 = lax.rem(my + 2, N_DEV)
        c2 = lax.rem(my + 1, N_DEV)
        c3 = my

        compute_chunk(c0)
        rdma0 = pltpu.make_async_remote_copy(
            src_ref=part_ref.at[pl.ds(c0 * chunk, chunk)],
            dst_ref=recv_ref.at[0],
            send_sem=send_sems.at[0], recv_sem=recv_sems.at[0],
            device_id=(right,), device_id_type=pl.DeviceIdType.MESH,
        )
        rdma0.start()
        compute_chunk(c1)
        rdma0.wait_recv()
        recv_ref[0, :, :] = recv_ref[0, :, :] + part_ref[pl.ds(c1 * chunk, chunk), :]

        rdma1 = pltpu.make_async_remote_copy(
            src_ref=recv_ref.at[0],
            dst_ref=recv_ref.at[1],
            send_sem=send_sems.at[1], recv_sem=recv_sems.at[1],
            device_id=(right,), device_id_type=pl.DeviceIdType.MESH,
        )
        rdma1.start()
        compute_chunk(c2)
        rdma1.wait_recv()
        recv_ref[1, :, :] = recv_ref[1, :, :] + part_ref[pl.ds(c2 * chunk, chunk), :]

        rdma2 = pltpu.make_async_remote_copy(
            src_ref=recv_ref.at[1],
            dst_ref=recv_ref.at[2],
            send_sem=send_sems.at[2], recv_sem=recv_sems.at[2],
            device_id=(right,), device_id_type=pl.DeviceIdType.MESH,
        )
        rdma2.start()
        compute_chunk(c3)
        rdma2.wait_recv()
        out_ref[:, :] = recv_ref[2, :, :] + part_ref[pl.ds(c3 * chunk, chunk), :]

        rdma0.wait_send()
        rdma1.wait_send()
        rdma2.wait_send()

    return pl.pallas_call(
        body,
        out_shape=jax.ShapeDtypeStruct((chunk, h), jnp.float32),
        in_specs=[pl.BlockSpec(memory_space=pltpu.VMEM)] * 4,
        out_specs=pl.BlockSpec(memory_space=pltpu.VMEM),
        scratch_shapes=[
            pltpu.VMEM((n, h), jnp.float32),
            pltpu.VMEM((N_DEV - 1, chunk, h), jnp.float32),
            pltpu.SemaphoreType.DMA((N_DEV - 1,)),
            pltpu.SemaphoreType.DMA((N_DEV - 1,)),
        ],
        compiler_params=pltpu.CompilerParams(
            collective_id=0,
            vmem_limit_bytes=100 * 1024 * 1024,
        ),
    )(x, router_W, route_idx, expert_W)


# device time: 64962 ns/iter; 1.8486x vs baseline; 1.5086x over previous
import jax
import jax.numpy as jnp
from jax import lax
from jax.experimental import pallas as pl
from jax.experimental.pallas import tpu as pltpu

N_DEV = 4


def kernel(x, router_W, route_idx, expert_W):
    n, d = x.shape
    e_loc, _, h = expert_W.shape
    n_exp = router_W.shape[1]
    chunk = n // N_DEV

    def body(x_ref, rw_ref, idx_ref, ew_ref, out_ref,
             part_ref, recv_ref, ewb_ref, send_sems, recv_sems):
        my = lax.axis_index("i")
        left = lax.rem(my + N_DEV - 1, N_DEV)
        right = lax.rem(my + 1, N_DEV)

        barrier = pltpu.get_barrier_semaphore()
        pl.semaphore_signal(barrier, inc=1, device_id=(left,),
                            device_id_type=pl.DeviceIdType.MESH)
        pl.semaphore_signal(barrier, inc=1, device_id=(right,),
                            device_id_type=pl.DeviceIdType.MESH)
        pl.semaphore_wait(barrier, 2)

        lids = my * e_loc + lax.broadcasted_iota(jnp.int32, (1, e_loc), 1)
        iota = lax.broadcasted_iota(jnp.int32, (chunk, n_exp), 1)

        ewb_ref[:, :, :] = ew_ref[:, :, :].astype(jnp.bfloat16)

        def compute_chunk(c):
            off = c * chunk
            xc = x_ref[pl.ds(off, chunk), :]
            scores = jnp.dot(xc, rw_ref[:, :],
                             preferred_element_type=jnp.float32)
            probs = jax.nn.softmax(scores, axis=-1)
            e0c = idx_ref[pl.ds(off, chunk), 0:1]
            e1c = idx_ref[pl.ds(off, chunk), 1:2]
            p0c = jnp.sum(jnp.where(iota == e0c, probs, 0.0),
                          axis=1, keepdims=True)
            p1c = jnp.sum(jnp.where(iota == e1c, probs, 0.0),
                          axis=1, keepdims=True)
            gates = (jnp.where(e0c == lids, p0c, 0.0)
                     + jnp.where(e1c == lids, p1c, 0.0)) / (p0c + p1c)
            acc = jnp.dot((xc * gates[:, 0:1]).astype(jnp.bfloat16),
                          ewb_ref[0], preferred_element_type=jnp.float32)
            for j in range(1, e_loc):
                acc = acc + jnp.dot((xc * gates[:, j:j + 1]).astype(jnp.bfloat16),
                                    ewb_ref[j],
                                    preferred_element_type=jnp.float32)
            part_ref[pl.ds(off, chunk), :] = acc.astype(jnp.bfloat16)

        c0 = lax.rem(my + 3, N_DEV)
        c1 = lax.rem(my + 2, N_DEV)
        c2 = lax.rem(my + 1, N_DEV)
        c3 = my

        compute_chunk(c0)
        rdma0 = pltpu.make_async_remote_copy(
            src_ref=part_ref.at[pl.ds(c0 * chunk, chunk)],
            dst_ref=recv_ref.at[0],
            send_sem=send_sems.at[0], recv_sem=recv_sems.at[0],
            device_id=(right,), device_id_type=pl.DeviceIdType.MESH,
        )
        rdma0.start()
        compute_chunk(c1)
        rdma0.wait_recv()
        recv_ref[0, :, :] = (
            recv_ref[0, :, :].astype(jnp.float32)
            + part_ref[pl.ds(c1 * chunk, chunk), :].astype(jnp.float32)
        ).astype(jnp.bfloat16)

        rdma1 = pltpu.make_async_remote_copy(
            src_ref=recv_ref.at[0],
            dst_ref=recv_ref.at[1],
            send_sem=send_sems.at[1], recv_sem=recv_sems.at[1],
            device_id=(right,), device_id_type=pl.DeviceIdType.MESH,
        )
        rdma1.start()
        compute_chunk(c2)
        rdma1.wait_recv()
        recv_ref[1, :, :] = (
            recv_ref[1, :, :].astype(jnp.float32)
            + part_ref[pl.ds(c2 * chunk, chunk), :].astype(jnp.float32)
        ).astype(jnp.bfloat16)

        rdma2 = pltpu.make_async_remote_copy(
            src_ref=recv_ref.at[1],
            dst_ref=recv_ref.at[2],
            send_sem=send_sems.at[2], recv_sem=recv_sems.at[2],
            device_id=(right,), device_id_type=pl.DeviceIdType.MESH,
        )
        rdma2.start()
        compute_chunk(c3)
        rdma2.wait_recv()
        out_ref[:, :] = (
            recv_ref[2, :, :].astype(jnp.float32)
            + part_ref[pl.ds(c3 * chunk, chunk), :].astype(jnp.float32)
        )

        rdma0.wait_send()
        rdma1.wait_send()
        rdma2.wait_send()

    return pl.pallas_call(
        body,
        out_shape=jax.ShapeDtypeStruct((chunk, h), jnp.float32),
        in_specs=[pl.BlockSpec(memory_space=pltpu.VMEM)] * 4,
        out_specs=pl.BlockSpec(memory_space=pltpu.VMEM),
        scratch_shapes=[
            pltpu.VMEM((n, h), jnp.bfloat16),
            pltpu.VMEM((N_DEV - 1, chunk, h), jnp.bfloat16),
            pltpu.VMEM((e_loc, d, h), jnp.bfloat16),
            pltpu.SemaphoreType.DMA((N_DEV - 1,)),
            pltpu.SemaphoreType.DMA((N_DEV - 1,)),
        ],
        compiler_params=pltpu.CompilerParams(
            collective_id=0,
            vmem_limit_bytes=100 * 1024 * 1024,
        ),
    )(x, router_W, route_idx, expert_W)


# device time: 55148 ns/iter; 2.1776x vs baseline; 1.1780x over previous
import jax
import jax.numpy as jnp
from jax import lax
from jax.experimental import pallas as pl
from jax.experimental.pallas import tpu as pltpu

N_DEV = 4


def kernel(x, router_W, route_idx, expert_W):
    n, d = x.shape
    e_loc, _, h = expert_W.shape
    n_exp = router_W.shape[1]
    chunk = n // N_DEV

    def body(x_ref, rw_ref, idx_ref, ew_ref, out_ref,
             part_ref, recv_ref, ewb_ref, send_sems, recv_sems):
        my = lax.axis_index("i")

        barrier = pltpu.get_barrier_semaphore()
        for k in range(1, N_DEV):
            peer = lax.rem(my + k, N_DEV)
            pl.semaphore_signal(barrier, inc=1, device_id=(peer,),
                                device_id_type=pl.DeviceIdType.MESH)
        pl.semaphore_wait(barrier, N_DEV - 1)

        lids = my * e_loc + lax.broadcasted_iota(jnp.int32, (1, e_loc), 1)
        iota = lax.broadcasted_iota(jnp.int32, (chunk, n_exp), 1)

        ewb_ref[:, :, :] = ew_ref[:, :, :].astype(jnp.bfloat16)

        def compute_chunk(c):
            off = c * chunk
            xc = x_ref[pl.ds(off, chunk), :]
            scores = jnp.dot(xc, rw_ref[:, :],
                             preferred_element_type=jnp.float32)
            probs = jax.nn.softmax(scores, axis=-1)
            e0c = idx_ref[pl.ds(off, chunk), 0:1]
            e1c = idx_ref[pl.ds(off, chunk), 1:2]
            p0c = jnp.sum(jnp.where(iota == e0c, probs, 0.0),
                          axis=1, keepdims=True)
            p1c = jnp.sum(jnp.where(iota == e1c, probs, 0.0),
                          axis=1, keepdims=True)
            gates = (jnp.where(e0c == lids, p0c, 0.0)
                     + jnp.where(e1c == lids, p1c, 0.0)) / (p0c + p1c)
            acc = jnp.dot((xc * gates[:, 0:1]).astype(jnp.bfloat16),
                          ewb_ref[0], preferred_element_type=jnp.float32)
            for j in range(1, e_loc):
                acc = acc + jnp.dot((xc * gates[:, j:j + 1]).astype(jnp.bfloat16),
                                    ewb_ref[j],
                                    preferred_element_type=jnp.float32)
            part_ref[pl.ds(off, chunk), :] = acc.astype(jnp.bfloat16)

        rdmas = []
        for k in range(1, N_DEV):
            c = lax.rem(my + k, N_DEV)
            compute_chunk(c)
            rdma = pltpu.make_async_remote_copy(
                src_ref=part_ref.at[pl.ds(c * chunk, chunk)],
                dst_ref=recv_ref.at[k - 1],
                send_sem=send_sems.at[k - 1],
                recv_sem=recv_sems.at[k - 1],
                device_id=(c,), device_id_type=pl.DeviceIdType.MESH,
            )
            rdma.start()
            rdmas.append(rdma)

        compute_chunk(my)

        for rdma in rdmas:
            rdma.wait_recv()
        out_ref[:, :] = (
            part_ref[pl.ds(my * chunk, chunk), :].astype(jnp.float32)
            + recv_ref[0, :, :].astype(jnp.float32)
            + recv_ref[1, :, :].astype(jnp.float32)
            + recv_ref[2, :, :].astype(jnp.float32)
        )

        for rdma in rdmas:
            rdma.wait_send()

    return pl.pallas_call(
        body,
        out_shape=jax.ShapeDtypeStruct((chunk, h), jnp.float32),
        in_specs=[pl.BlockSpec(memory_space=pltpu.VMEM)] * 4,
        out_specs=pl.BlockSpec(memory_space=pltpu.VMEM),
        scratch_shapes=[
            pltpu.VMEM((n, h), jnp.bfloat16),
            pltpu.VMEM((N_DEV - 1, chunk, h), jnp.bfloat16),
            pltpu.VMEM((e_loc, d, h), jnp.bfloat16),
            pltpu.SemaphoreType.DMA((N_DEV - 1,)),
            pltpu.SemaphoreType.DMA((N_DEV - 1,)),
        ],
        compiler_params=pltpu.CompilerParams(
            collective_id=0,
            vmem_limit_bytes=100 * 1024 * 1024,
        ),
    )(x, router_W, route_idx, expert_W)
